# Initial kernel scaffold; baseline (speedup 1.0000x reference)
import jax
import jax.numpy as jnp
from jax.experimental import pallas as pl


def kernel(x, edge_index, Wl1, bl1, Wr1, Wl2, bl2, Wr2, Wd, bd):
    raise NotImplementedError("write your pallas kernel here")


# speedup vs baseline: 6.9692x; 6.9692x over previous
import functools

import jax
import jax.numpy as jnp
from jax import lax
from jax.experimental import pallas as pl
from jax.experimental.pallas import tpu as pltpu
from jax.experimental.pallas import tpu_sc as plsc

N = 10000
E = 320000
IN_DIM = 128
HIDDEN = 128
LATENT = 64

NC = 2
NS = 16
NW = NC * NS
BLK = 128
NBLKS = E // BLK
BASE_BLKS = NBLKS // NW
EXTRA = NBLKS - BASE_BLKS * NW
ROWS_A = 624
ROWS_LAST = N - ROWS_A * (NS - 1)

_MESH = plsc.VectorSubcoreMesh(
    core_axis_name="c", subcore_axis_name="s", num_cores=NC, num_subcores=NS)


def _make_segsum(D, with_deg):
  out_type = [jax.ShapeDtypeStruct((NC, N, D), jnp.float32)]
  if with_deg:
    out_type.append(jax.ShapeDtypeStruct((NC, N, 16), jnp.float32))
  scratch = [
      pltpu.VMEM((BLK,), jnp.int32),
      pltpu.VMEM((BLK,), jnp.int32),
      pltpu.VMEM((BLK, D), jnp.float32),
      pltpu.VMEM_SHARED((N, D), jnp.float32),
      pltpu.SemaphoreType.DMA,
  ]
  if with_deg:
    scratch += [
        pltpu.VMEM((BLK, 16), jnp.float32),
        pltpu.VMEM_SHARED((N, 16), jnp.float32),
    ]

  def body(*refs):
    if with_deg:
      (p_hbm, src_hbm, dst_hbm, z_hbm, ones_hbm, z16_hbm,
       part_hbm, degp_hbm,
       sidx, didx, rows, acc, gsem, ones_v, dacc) = refs
    else:
      (p_hbm, src_hbm, dst_hbm, z_hbm,
       part_hbm,
       sidx, didx, rows, acc, gsem) = refs
    c = lax.axis_index("c")
    s = lax.axis_index("s")
    w = c * NS + s
    r0 = s * ROWS_A

    def tile_slices(fn):
      @pl.when(s < NS - 1)
      def _():
        fn(r0, ROWS_A)

      @pl.when(s == NS - 1)
      def _():
        fn(ROWS_A * (NS - 1), ROWS_LAST)

    tile_slices(lambda o, n: pltpu.sync_copy(z_hbm.at[pl.ds(o, n)],
                                             acc.at[pl.ds(o, n)]))
    if with_deg:
      tile_slices(lambda o, n: pltpu.sync_copy(z16_hbm.at[pl.ds(o, n)],
                                               dacc.at[pl.ds(o, n)]))
      pltpu.sync_copy(ones_hbm, ones_v)
    plsc.subcore_barrier()

    nblk = BASE_BLKS + jnp.where(w < EXTRA, 1, 0)
    blk0 = BASE_BLKS * w + jnp.minimum(w, EXTRA)

    def step(g, carry):
      base = (blk0 + g) * BLK
      pltpu.sync_copy(src_hbm.at[pl.ds(base, BLK)], sidx)
      pltpu.sync_copy(dst_hbm.at[pl.ds(base, BLK)], didx)
      pltpu.async_copy(p_hbm.at[sidx], rows, gsem).wait()
      pltpu.sync_copy(rows, acc.at[didx], add=True)
      if with_deg:
        pltpu.sync_copy(ones_v, dacc.at[didx], add=True)
      return carry

    lax.fori_loop(0, nblk, step, 0)
    plsc.subcore_barrier()

    tile_slices(lambda o, n: pltpu.sync_copy(acc.at[pl.ds(o, n)],
                                             part_hbm.at[c, pl.ds(o, n)]))
    if with_deg:
      tile_slices(lambda o, n: pltpu.sync_copy(dacc.at[pl.ds(o, n)],
                                               degp_hbm.at[c, pl.ds(o, n)]))

  params = pltpu.CompilerParams(use_tc_tiling_on_sc=False)
  return pl.kernel(body, out_type=tuple(out_type), mesh=_MESH,
                   scratch_types=scratch, compiler_params=params)


_segsum_deg = _make_segsum(HIDDEN, True)
_segsum_l2 = _make_segsum(LATENT, False)


ROWS_TC = 1000


def _tc_pre(x, Wl1, Wr1):
  def body(x_ref, wl_ref, wr_ref, p1_ref, r1_ref):
    xb = x_ref[...]
    p1_ref[...] = lax.dot_general(xb, wl_ref[...], (((1,), (1,)), ((), ())),
                                  preferred_element_type=jnp.float32)
    r1_ref[...] = lax.dot_general(xb, wr_ref[...], (((1,), (1,)), ((), ())),
                                  preferred_element_type=jnp.float32)
  grid = (N // ROWS_TC,)
  return pl.pallas_call(
      body,
      grid=grid,
      in_specs=[
          pl.BlockSpec((ROWS_TC, IN_DIM), lambda i: (i, 0)),
          pl.BlockSpec((HIDDEN, IN_DIM), lambda i: (0, 0)),
          pl.BlockSpec((HIDDEN, IN_DIM), lambda i: (0, 0)),
      ],
      out_specs=[
          pl.BlockSpec((ROWS_TC, HIDDEN), lambda i: (i, 0)),
          pl.BlockSpec((ROWS_TC, HIDDEN), lambda i: (i, 0)),
      ],
      out_shape=[
          jax.ShapeDtypeStruct((N, HIDDEN), jnp.float32),
          jax.ShapeDtypeStruct((N, HIDDEN), jnp.float32),
      ],
  )(x, Wl1, Wr1)


def _tc_mid(s1p, degp, bl1, r1, Wl2, Wr2):
  def body(s1p_ref, degp_ref, bl1_ref, r1_ref, wl2_ref, wr2_ref,
           p2_ref, r2_ref):
    ssum = s1p_ref[0] + s1p_ref[1]
    dcol = degp_ref[0, :, 0] + degp_ref[1, :, 0]
    deg = jnp.maximum(dcol, 1.0)[:, None]
    h = jnp.maximum(ssum / deg + bl1_ref[...] + r1_ref[...], 0.0)
    p2_ref[...] = lax.dot_general(h, wl2_ref[...], (((1,), (1,)), ((), ())),
                                  preferred_element_type=jnp.float32)
    r2_ref[...] = lax.dot_general(h, wr2_ref[...], (((1,), (1,)), ((), ())),
                                  preferred_element_type=jnp.float32)
  grid = (N // ROWS_TC,)
  return pl.pallas_call(
      body,
      grid=grid,
      in_specs=[
          pl.BlockSpec((NC, ROWS_TC, HIDDEN), lambda i: (0, i, 0)),
          pl.BlockSpec((NC, ROWS_TC, 16), lambda i: (0, i, 0)),
          pl.BlockSpec((1, HIDDEN), lambda i: (0, 0)),
          pl.BlockSpec((ROWS_TC, HIDDEN), lambda i: (i, 0)),
          pl.BlockSpec((LATENT, HIDDEN), lambda i: (0, 0)),
          pl.BlockSpec((LATENT, HIDDEN), lambda i: (0, 0)),
      ],
      out_specs=[
          pl.BlockSpec((ROWS_TC, LATENT), lambda i: (i, 0)),
          pl.BlockSpec((ROWS_TC, LATENT), lambda i: (i, 0)),
      ],
      out_shape=[
          jax.ShapeDtypeStruct((N, LATENT), jnp.float32),
          jax.ShapeDtypeStruct((N, LATENT), jnp.float32),
      ],
  )(s1p, degp, bl1, r1, Wl2, Wr2)


def _tc_post(s2p, degp, bl2, r2, Wd, bd):
  def body(s2p_ref, degp_ref, bl2_ref, r2_ref, wd_ref, bd_ref,
           z_ref, xh_ref):
    ssum = s2p_ref[0] + s2p_ref[1]
    dcol = degp_ref[0, :, 0] + degp_ref[1, :, 0]
    deg = jnp.maximum(dcol, 1.0)[:, None]
    z = ssum / deg + bl2_ref[...] + r2_ref[...]
    z_ref[...] = z
    xh_ref[...] = lax.dot_general(z, wd_ref[...], (((1,), (1,)), ((), ())),
                                  preferred_element_type=jnp.float32) + bd_ref[...]
  grid = (N // ROWS_TC,)
  return pl.pallas_call(
      body,
      grid=grid,
      in_specs=[
          pl.BlockSpec((NC, ROWS_TC, LATENT), lambda i: (0, i, 0)),
          pl.BlockSpec((NC, ROWS_TC, 16), lambda i: (0, i, 0)),
          pl.BlockSpec((1, LATENT), lambda i: (0, 0)),
          pl.BlockSpec((ROWS_TC, LATENT), lambda i: (i, 0)),
          pl.BlockSpec((IN_DIM, LATENT), lambda i: (0, 0)),
          pl.BlockSpec((1, IN_DIM), lambda i: (0, 0)),
      ],
      out_specs=[
          pl.BlockSpec((ROWS_TC, LATENT), lambda i: (i, 0)),
          pl.BlockSpec((ROWS_TC, IN_DIM), lambda i: (i, 0)),
      ],
      out_shape=[
          jax.ShapeDtypeStruct((N, LATENT), jnp.float32),
          jax.ShapeDtypeStruct((N, IN_DIM), jnp.float32),
      ],
  )(s2p, degp, bl2, r2, Wd, bd)


def kernel(x, edge_index, Wl1, bl1, Wr1, Wl2, bl2, Wr2, Wd, bd):
  src = edge_index[0].astype(jnp.int32)
  dst = edge_index[1].astype(jnp.int32)

  ones16 = jnp.zeros((BLK, 16), jnp.float32).at[:, 0].set(1.0)
  z128 = jnp.zeros((N, HIDDEN), jnp.float32)
  z64 = jnp.zeros((N, LATENT), jnp.float32)
  z16 = jnp.zeros((N, 16), jnp.float32)

  p1, r1 = _tc_pre(x, Wl1, Wr1)
  s1p, degp = _segsum_deg(p1, src, dst, z128, ones16, z16)
  p2, r2 = _tc_mid(s1p, degp, bl1.reshape(1, HIDDEN), r1, Wl2, Wr2)
  (s2p,) = _segsum_l2(p2, src, dst, z64)
  z, x_hat = _tc_post(s2p, degp, bl2.reshape(1, LATENT), r2, Wd,
                      bd.reshape(1, IN_DIM))
  return (z, x_hat)


# speedup vs baseline: 11.1424x; 1.5988x over previous
import jax
import jax.numpy as jnp
from jax import lax
from jax.experimental import pallas as pl
from jax.experimental.pallas import tpu as pltpu
from jax.experimental.pallas import tpu_sc as plsc

N = 10000
E = 320000
IN_DIM = 128
HIDDEN = 128
LATENT = 64

NC = 2
NS = 16
BLK = 128
NBLKS = E // BLK
BASE_BLKS = NBLKS // NS
EXTRA = NBLKS - BASE_BLKS * NS
MAXB = BASE_BLKS + 1
ROWS_A = 624
ROWS_LAST = N - ROWS_A * (NS - 1)

_MESH = plsc.VectorSubcoreMesh(
    core_axis_name="c", subcore_axis_name="s", num_cores=NC, num_subcores=NS)


def _make_segsum(DH, with_deg):
  out_type = [jax.ShapeDtypeStruct((NC, N, DH), jnp.float32)]
  if with_deg:
    out_type.append(jax.ShapeDtypeStruct((N, 16), jnp.float32))
  scratch = [
      pltpu.VMEM((MAXB, BLK), jnp.int32),
      pltpu.VMEM((MAXB, BLK), jnp.int32),
      pltpu.VMEM((BLK, DH), jnp.float32),
      pltpu.VMEM((BLK, DH), jnp.float32),
      pltpu.VMEM_SHARED((N, DH), jnp.float32),
      pltpu.SemaphoreType.DMA,
      pltpu.SemaphoreType.DMA,
  ]
  if with_deg:
    scratch += [
        pltpu.VMEM((BLK, 16), jnp.float32),
        pltpu.VMEM_SHARED((N, 16), jnp.float32),
    ]

  def body(*refs):
    if with_deg:
      (pa_hbm, pb_hbm, src_hbm, dst_hbm, z_hbm, ones_hbm, z1_hbm,
       part_hbm, deg_hbm,
       sidx, didx, rows0, rows1, acc, gsem0, gsem1, ones_v, dacc) = refs
    else:
      (pa_hbm, pb_hbm, src_hbm, dst_hbm, z_hbm,
       part_hbm,
       sidx, didx, rows0, rows1, acc, gsem0, gsem1) = refs
    c = lax.axis_index("c")
    s = lax.axis_index("s")
    r0 = s * ROWS_A

    def tile_slices(fn):
      @pl.when(s < NS - 1)
      def _():
        fn(r0, ROWS_A)

      @pl.when(s == NS - 1)
      def _():
        fn(ROWS_A * (NS - 1), ROWS_LAST)

    tile_slices(lambda o, n: pltpu.sync_copy(z_hbm.at[pl.ds(o, n)],
                                             acc.at[pl.ds(o, n)]))
    if with_deg:
      @pl.when(c == 0)
      def _():
        tile_slices(lambda o, n: pltpu.sync_copy(z1_hbm.at[pl.ds(o, n)],
                                                 dacc.at[pl.ds(o, n)]))
        pltpu.sync_copy(ones_hbm, ones_v)
    plsc.subcore_barrier()

    nblk = BASE_BLKS + jnp.where(s < EXTRA, 1, 0)
    blk0 = BASE_BLKS * s + jnp.minimum(s, EXTRA)

    @pl.when(s < EXTRA)
    def _():
      pltpu.sync_copy(src_hbm.at[pl.ds(blk0, MAXB)], sidx)
      pltpu.sync_copy(dst_hbm.at[pl.ds(blk0, MAXB)], didx)

    @pl.when(s >= EXTRA)
    def _():
      pltpu.sync_copy(src_hbm.at[pl.ds(blk0, BASE_BLKS)],
                      sidx.at[pl.ds(0, BASE_BLKS)])
      pltpu.sync_copy(dst_hbm.at[pl.ds(blk0, BASE_BLKS)],
                      didx.at[pl.ds(0, BASE_BLKS)])

    def on_buf_src(g, fn):
      @pl.when((lax.rem(g, 2) == 0) & (c == 0))
      def _():
        fn(pa_hbm, rows0, gsem0)

      @pl.when((lax.rem(g, 2) == 1) & (c == 0))
      def _():
        fn(pa_hbm, rows1, gsem1)

      @pl.when((lax.rem(g, 2) == 0) & (c == 1))
      def _():
        fn(pb_hbm, rows0, gsem0)

      @pl.when((lax.rem(g, 2) == 1) & (c == 1))
      def _():
        fn(pb_hbm, rows1, gsem1)

    def gather_start(g):
      on_buf_src(g, lambda p, buf, sem: pltpu.async_copy(
          p.at[sidx.at[g]], buf, sem))

    gather_start(0)

    def step(g, carry):
      @pl.when(g + 1 < nblk)
      def _():
        gather_start(g + 1)

      def drain_and_scatter(p, buf, sem):
        pltpu.make_async_copy(p.at[sidx.at[g]], buf, sem).wait()
        pltpu.sync_copy(buf, acc.at[didx.at[g]], add=True)

      on_buf_src(g, drain_and_scatter)
      if with_deg:
        @pl.when(c == 0)
        def _():
          pltpu.sync_copy(ones_v, dacc.at[didx.at[g]], add=True)
      return carry

    lax.fori_loop(0, nblk, step, 0)
    plsc.subcore_barrier()

    tile_slices(lambda o, n: pltpu.sync_copy(acc.at[pl.ds(o, n)],
                                             part_hbm.at[c, pl.ds(o, n)]))
    if with_deg:
      @pl.when(c == 0)
      def _():
        tile_slices(lambda o, n: pltpu.sync_copy(dacc.at[pl.ds(o, n)],
                                                 deg_hbm.at[pl.ds(o, n)]))

  params = pltpu.CompilerParams(use_tc_tiling_on_sc=False)
  return pl.kernel(body, out_type=tuple(out_type), mesh=_MESH,
                   scratch_types=scratch, compiler_params=params)


_segsum_deg = _make_segsum(HIDDEN // 2, True)
_segsum_l2 = _make_segsum(LATENT // 2, False)


ROWS_TC = 1000


def _tc_pre(x, Wl1a, Wl1b, Wr1):
  def body(x_ref, wla_ref, wlb_ref, wr_ref, p1a_ref, p1b_ref, r1_ref):
    xb = x_ref[...]
    dn = (((1,), (1,)), ((), ()))
    p1a_ref[...] = lax.dot_general(xb, wla_ref[...], dn,
                                   preferred_element_type=jnp.float32)
    p1b_ref[...] = lax.dot_general(xb, wlb_ref[...], dn,
                                   preferred_element_type=jnp.float32)
    r1_ref[...] = lax.dot_general(xb, wr_ref[...], dn,
                                  preferred_element_type=jnp.float32)
  grid = (N // ROWS_TC,)
  H2 = HIDDEN // 2
  return pl.pallas_call(
      body,
      grid=grid,
      in_specs=[
          pl.BlockSpec((ROWS_TC, IN_DIM), lambda i: (i, 0)),
          pl.BlockSpec((H2, IN_DIM), lambda i: (0, 0)),
          pl.BlockSpec((H2, IN_DIM), lambda i: (0, 0)),
          pl.BlockSpec((HIDDEN, IN_DIM), lambda i: (0, 0)),
      ],
      out_specs=[
          pl.BlockSpec((ROWS_TC, H2), lambda i: (i, 0)),
          pl.BlockSpec((ROWS_TC, H2), lambda i: (i, 0)),
          pl.BlockSpec((ROWS_TC, HIDDEN), lambda i: (i, 0)),
      ],
      out_shape=[
          jax.ShapeDtypeStruct((N, H2), jnp.float32),
          jax.ShapeDtypeStruct((N, H2), jnp.float32),
          jax.ShapeDtypeStruct((N, HIDDEN), jnp.float32),
      ],
  )(x, Wl1a, Wl1b, Wr1)


def _tc_mid(s1p, deg, bl1, r1, Wl2a, Wl2b, Wr2):
  H2 = HIDDEN // 2
  L2 = LATENT // 2

  def body(s1p_ref, deg_ref, bl1_ref, r1_ref, wla_ref, wlb_ref, wr_ref,
           p2a_ref, p2b_ref, r2_ref):
    ssum = jnp.concatenate([s1p_ref[0], s1p_ref[1]], axis=1)
    d = jnp.maximum(deg_ref[...][:, 0:1], 1.0)
    h = jnp.maximum(ssum / d + bl1_ref[...] + r1_ref[...], 0.0)
    dn = (((1,), (1,)), ((), ()))
    p2a_ref[...] = lax.dot_general(h, wla_ref[...], dn,
                                   preferred_element_type=jnp.float32)
    p2b_ref[...] = lax.dot_general(h, wlb_ref[...], dn,
                                   preferred_element_type=jnp.float32)
    r2_ref[...] = lax.dot_general(h, wr_ref[...], dn,
                                  preferred_element_type=jnp.float32)
  grid = (N // ROWS_TC,)
  return pl.pallas_call(
      body,
      grid=grid,
      in_specs=[
          pl.BlockSpec((NC, ROWS_TC, H2), lambda i: (0, i, 0)),
          pl.BlockSpec((ROWS_TC, 16), lambda i: (i, 0)),
          pl.BlockSpec((1, HIDDEN), lambda i: (0, 0)),
          pl.BlockSpec((ROWS_TC, HIDDEN), lambda i: (i, 0)),
          pl.BlockSpec((L2, HIDDEN), lambda i: (0, 0)),
          pl.BlockSpec((L2, HIDDEN), lambda i: (0, 0)),
          pl.BlockSpec((LATENT, HIDDEN), lambda i: (0, 0)),
      ],
      out_specs=[
          pl.BlockSpec((ROWS_TC, L2), lambda i: (i, 0)),
          pl.BlockSpec((ROWS_TC, L2), lambda i: (i, 0)),
          pl.BlockSpec((ROWS_TC, LATENT), lambda i: (i, 0)),
      ],
      out_shape=[
          jax.ShapeDtypeStruct((N, L2), jnp.float32),
          jax.ShapeDtypeStruct((N, L2), jnp.float32),
          jax.ShapeDtypeStruct((N, LATENT), jnp.float32),
      ],
  )(s1p, deg, bl1, r1, Wl2a, Wl2b, Wr2)


def _tc_post(s2p, deg, bl2, r2, Wd, bd):
  L2 = LATENT // 2

  def body(s2p_ref, deg_ref, bl2_ref, r2_ref, wd_ref, bd_ref,
           z_ref, xh_ref):
    ssum = jnp.concatenate([s2p_ref[0], s2p_ref[1]], axis=1)
    d = jnp.maximum(deg_ref[...][:, 0:1], 1.0)
    z = ssum / d + bl2_ref[...] + r2_ref[...]
    z_ref[...] = z
    xh_ref[...] = lax.dot_general(z, wd_ref[...], (((1,), (1,)), ((), ())),
                                  preferred_element_type=jnp.float32) + bd_ref[...]
  grid = (N // ROWS_TC,)
  return pl.pallas_call(
      body,
      grid=grid,
      in_specs=[
          pl.BlockSpec((NC, ROWS_TC, L2), lambda i: (0, i, 0)),
          pl.BlockSpec((ROWS_TC, 16), lambda i: (i, 0)),
          pl.BlockSpec((1, LATENT), lambda i: (0, 0)),
          pl.BlockSpec((ROWS_TC, LATENT), lambda i: (i, 0)),
          pl.BlockSpec((IN_DIM, LATENT), lambda i: (0, 0)),
          pl.BlockSpec((1, IN_DIM), lambda i: (0, 0)),
      ],
      out_specs=[
          pl.BlockSpec((ROWS_TC, LATENT), lambda i: (i, 0)),
          pl.BlockSpec((ROWS_TC, IN_DIM), lambda i: (i, 0)),
      ],
      out_shape=[
          jax.ShapeDtypeStruct((N, LATENT), jnp.float32),
          jax.ShapeDtypeStruct((N, IN_DIM), jnp.float32),
      ],
  )(s2p, deg, bl2, r2, Wd, bd)


def kernel(x, edge_index, Wl1, bl1, Wr1, Wl2, bl2, Wr2, Wd, bd):
  src = edge_index[0].astype(jnp.int32).reshape(NBLKS, BLK)
  dst = edge_index[1].astype(jnp.int32).reshape(NBLKS, BLK)

  H2 = HIDDEN // 2
  L2 = LATENT // 2
  ones16 = jnp.zeros((BLK, 16), jnp.float32).at[:, 0].set(1.0)
  zh = jnp.zeros((N, H2), jnp.float32)
  zl = jnp.zeros((N, L2), jnp.float32)
  z16 = jnp.zeros((N, 16), jnp.float32)

  p1a, p1b, r1 = _tc_pre(x, Wl1[:H2], Wl1[H2:], Wr1)
  s1p, deg = _segsum_deg(p1a, p1b, src, dst, zh, ones16, z16)
  p2a, p2b, r2 = _tc_mid(s1p, deg, bl1.reshape(1, HIDDEN), r1,
                         Wl2[:L2], Wl2[L2:], Wr2)
  (s2p,) = _segsum_l2(p2a, p2b, src, dst, zl)
  z, x_hat = _tc_post(s2p, deg, bl2.reshape(1, LATENT), r2, Wd,
                      bd.reshape(1, IN_DIM))
  return (z, x_hat)


# speedup vs baseline: 13.2248x; 1.1869x over previous
import jax
import jax.numpy as jnp
from jax import lax
from jax.experimental import pallas as pl
from jax.experimental.pallas import tpu as pltpu
from jax.experimental.pallas import tpu_sc as plsc

N = 10000
E = 320000
IN_DIM = 128
HIDDEN = 128
LATENT = 64

NC = 2
NS = 16
BLK = 128
NBLKS = E // BLK
BASE_BLKS = NBLKS // NS
EXTRA = NBLKS - BASE_BLKS * NS
MAXB = BASE_BLKS + 1
ROWS_A = 624
ROWS_LAST = N - ROWS_A * (NS - 1)

_MESH = plsc.VectorSubcoreMesh(
    core_axis_name="c", subcore_axis_name="s", num_cores=NC, num_subcores=NS)


def _make_segsum(DH, with_deg):
  NBUF = 4
  out_type = [jax.ShapeDtypeStruct((NC, N, DH), jnp.float32)]
  if with_deg:
    out_type.append(jax.ShapeDtypeStruct((NC, N, 16), jnp.float32))
  scratch = [
      pltpu.VMEM((MAXB, BLK), jnp.int32),
      pltpu.VMEM((MAXB, BLK), jnp.int32),
  ]
  scratch += [pltpu.VMEM((BLK, DH), jnp.float32) for _ in range(NBUF)]
  scratch += [
      pltpu.VMEM_SHARED((N, DH), jnp.float32),
  ]
  scratch += [pltpu.SemaphoreType.DMA for _ in range(2 * NBUF)]
  if with_deg:
    scratch += [
        pltpu.VMEM((BLK, 16), jnp.float32),
        pltpu.VMEM_SHARED((N, 16), jnp.float32),
    ]

  def body(*refs):
    if with_deg:
      (pa_hbm, pb_hbm, src_hbm, dst_hbm, z_hbm, ones_hbm, z1_hbm,
       part_hbm, deg_hbm,
       sidx, didx, *rest) = refs
      bufs = rest[:NBUF]
      acc = rest[NBUF]
      gsems = rest[NBUF + 1:2 * NBUF + 1]
      ssems = rest[2 * NBUF + 1:3 * NBUF + 1]
      ones_v, dacc = rest[3 * NBUF + 1:]
    else:
      (pa_hbm, pb_hbm, src_hbm, dst_hbm, z_hbm,
       part_hbm,
       sidx, didx, *rest) = refs
      bufs = rest[:NBUF]
      acc = rest[NBUF]
      gsems = rest[NBUF + 1:2 * NBUF + 1]
      ssems = rest[2 * NBUF + 1:3 * NBUF + 1]
    c = lax.axis_index("c")
    s = lax.axis_index("s")
    r0 = s * ROWS_A

    def tile_slices(fn):
      @pl.when(s < NS - 1)
      def _():
        fn(r0, ROWS_A)

      @pl.when(s == NS - 1)
      def _():
        fn(ROWS_A * (NS - 1), ROWS_LAST)

    tile_slices(lambda o, n: pltpu.sync_copy(z_hbm.at[pl.ds(o, n)],
                                             acc.at[pl.ds(o, n)]))
    if with_deg:
      tile_slices(lambda o, n: pltpu.sync_copy(z1_hbm.at[pl.ds(o, n)],
                                               dacc.at[pl.ds(o, n)]))
      pltpu.sync_copy(ones_hbm, ones_v)
    plsc.subcore_barrier()

    nblk = BASE_BLKS + jnp.where(s < EXTRA, 1, 0)
    blk0 = BASE_BLKS * s + jnp.minimum(s, EXTRA)

    @pl.when(s < EXTRA)
    def _():
      pltpu.sync_copy(src_hbm.at[pl.ds(blk0, MAXB)], sidx)
      pltpu.sync_copy(dst_hbm.at[pl.ds(blk0, MAXB)], didx)

    @pl.when(s >= EXTRA)
    def _():
      pltpu.sync_copy(src_hbm.at[pl.ds(blk0, BASE_BLKS)],
                      sidx.at[pl.ds(0, BASE_BLKS)])
      pltpu.sync_copy(dst_hbm.at[pl.ds(blk0, BASE_BLKS)],
                      didx.at[pl.ds(0, BASE_BLKS)])

    def on_slot(g, fn):
      for b in range(NBUF):
        for cc in range(NC):
          @pl.when((lax.rem(g, NBUF) == b) & (c == cc))
          def _(b=b, cc=cc):
            fn((pa_hbm, pb_hbm)[cc], bufs[b], gsems[b], ssems[b])

    def gather_start(g):
      on_slot(g, lambda p, buf, gsem, ssem: pltpu.async_copy(
          p.at[sidx.at[g]], buf, gsem))

    def scatter_wait(g):
      on_slot(g, lambda p, buf, gsem, ssem: pltpu.make_async_copy(
          buf, acc.at[didx.at[g]], ssem).wait())

    P = NBUF // 2
    for k in range(P):
      gather_start(k)

    def step(g, carry):
      @pl.when((g + P < nblk) & (g + P - NBUF >= 0))
      def _():
        scatter_wait(g + P - NBUF)

      @pl.when(g + P < nblk)
      def _():
        gather_start(g + P)

      def drain_and_scatter(p, buf, gsem, ssem):
        pltpu.make_async_copy(p.at[sidx.at[g]], buf, gsem).wait()
        pltpu.async_copy(buf, acc.at[didx.at[g]], ssem, add=True)

      on_slot(g, drain_and_scatter)
      if with_deg:
        @pl.when(lax.rem(g, 2) == c)
        def _():
          pltpu.sync_copy(ones_v, dacc.at[didx.at[g]], add=True)
      return carry

    lax.fori_loop(0, nblk, step, 0)
    for k in range(NBUF):
      @pl.when(nblk - NBUF + k >= 0)
      def _(k=k):
        scatter_wait(nblk - NBUF + k)
    plsc.subcore_barrier()

    tile_slices(lambda o, n: pltpu.sync_copy(acc.at[pl.ds(o, n)],
                                             part_hbm.at[c, pl.ds(o, n)]))
    if with_deg:
      tile_slices(lambda o, n: pltpu.sync_copy(dacc.at[pl.ds(o, n)],
                                               deg_hbm.at[c, pl.ds(o, n)]))

  params = pltpu.CompilerParams(use_tc_tiling_on_sc=False)
  return pl.kernel(body, out_type=tuple(out_type), mesh=_MESH,
                   scratch_types=scratch, compiler_params=params)


_segsum_deg = _make_segsum(HIDDEN // 2, True)
_segsum_l2 = _make_segsum(LATENT // 2, False)


ROWS_TC = 1000


def _tc_pre(x, Wl1a, Wl1b, Wr1):
  def body(x_ref, wla_ref, wlb_ref, wr_ref, p1a_ref, p1b_ref, r1_ref):
    xb = x_ref[...]
    dn = (((1,), (1,)), ((), ()))
    p1a_ref[...] = lax.dot_general(xb, wla_ref[...], dn,
                                   preferred_element_type=jnp.float32)
    p1b_ref[...] = lax.dot_general(xb, wlb_ref[...], dn,
                                   preferred_element_type=jnp.float32)
    r1_ref[...] = lax.dot_general(xb, wr_ref[...], dn,
                                  preferred_element_type=jnp.float32)
  grid = (N // ROWS_TC,)
  H2 = HIDDEN // 2
  return pl.pallas_call(
      body,
      grid=grid,
      in_specs=[
          pl.BlockSpec((ROWS_TC, IN_DIM), lambda i: (i, 0)),
          pl.BlockSpec((H2, IN_DIM), lambda i: (0, 0)),
          pl.BlockSpec((H2, IN_DIM), lambda i: (0, 0)),
          pl.BlockSpec((HIDDEN, IN_DIM), lambda i: (0, 0)),
      ],
      out_specs=[
          pl.BlockSpec((ROWS_TC, H2), lambda i: (i, 0)),
          pl.BlockSpec((ROWS_TC, H2), lambda i: (i, 0)),
          pl.BlockSpec((ROWS_TC, HIDDEN), lambda i: (i, 0)),
      ],
      out_shape=[
          jax.ShapeDtypeStruct((N, H2), jnp.float32),
          jax.ShapeDtypeStruct((N, H2), jnp.float32),
          jax.ShapeDtypeStruct((N, HIDDEN), jnp.float32),
      ],
  )(x, Wl1a, Wl1b, Wr1)


def _tc_mid(s1p, deg, bl1, r1, Wl2a, Wl2b, Wr2):
  H2 = HIDDEN // 2
  L2 = LATENT // 2

  def body(s1p_ref, deg_ref, bl1_ref, r1_ref, wla_ref, wlb_ref, wr_ref,
           p2a_ref, p2b_ref, r2_ref):
    ssum = jnp.concatenate([s1p_ref[0], s1p_ref[1]], axis=1)
    d = jnp.maximum(deg_ref[0, :, 0:1] + deg_ref[1, :, 0:1], 1.0)
    h = jnp.maximum(ssum / d + bl1_ref[...] + r1_ref[...], 0.0)
    dn = (((1,), (1,)), ((), ()))
    p2a_ref[...] = lax.dot_general(h, wla_ref[...], dn,
                                   preferred_element_type=jnp.float32)
    p2b_ref[...] = lax.dot_general(h, wlb_ref[...], dn,
                                   preferred_element_type=jnp.float32)
    r2_ref[...] = lax.dot_general(h, wr_ref[...], dn,
                                  preferred_element_type=jnp.float32)
  grid = (N // ROWS_TC,)
  return pl.pallas_call(
      body,
      grid=grid,
      in_specs=[
          pl.BlockSpec((NC, ROWS_TC, H2), lambda i: (0, i, 0)),
          pl.BlockSpec((NC, ROWS_TC, 16), lambda i: (0, i, 0)),
          pl.BlockSpec((1, HIDDEN), lambda i: (0, 0)),
          pl.BlockSpec((ROWS_TC, HIDDEN), lambda i: (i, 0)),
          pl.BlockSpec((L2, HIDDEN), lambda i: (0, 0)),
          pl.BlockSpec((L2, HIDDEN), lambda i: (0, 0)),
          pl.BlockSpec((LATENT, HIDDEN), lambda i: (0, 0)),
      ],
      out_specs=[
          pl.BlockSpec((ROWS_TC, L2), lambda i: (i, 0)),
          pl.BlockSpec((ROWS_TC, L2), lambda i: (i, 0)),
          pl.BlockSpec((ROWS_TC, LATENT), lambda i: (i, 0)),
      ],
      out_shape=[
          jax.ShapeDtypeStruct((N, L2), jnp.float32),
          jax.ShapeDtypeStruct((N, L2), jnp.float32),
          jax.ShapeDtypeStruct((N, LATENT), jnp.float32),
      ],
  )(s1p, deg, bl1, r1, Wl2a, Wl2b, Wr2)


def _tc_post(s2p, deg, bl2, r2, Wd, bd):
  L2 = LATENT // 2

  def body(s2p_ref, deg_ref, bl2_ref, r2_ref, wd_ref, bd_ref,
           z_ref, xh_ref):
    ssum = jnp.concatenate([s2p_ref[0], s2p_ref[1]], axis=1)
    d = jnp.maximum(deg_ref[0, :, 0:1] + deg_ref[1, :, 0:1], 1.0)
    z = ssum / d + bl2_ref[...] + r2_ref[...]
    z_ref[...] = z
    xh_ref[...] = lax.dot_general(z, wd_ref[...], (((1,), (1,)), ((), ())),
                                  preferred_element_type=jnp.float32) + bd_ref[...]
  grid = (N // ROWS_TC,)
  return pl.pallas_call(
      body,
      grid=grid,
      in_specs=[
          pl.BlockSpec((NC, ROWS_TC, L2), lambda i: (0, i, 0)),
          pl.BlockSpec((NC, ROWS_TC, 16), lambda i: (0, i, 0)),
          pl.BlockSpec((1, LATENT), lambda i: (0, 0)),
          pl.BlockSpec((ROWS_TC, LATENT), lambda i: (i, 0)),
          pl.BlockSpec((IN_DIM, LATENT), lambda i: (0, 0)),
          pl.BlockSpec((1, IN_DIM), lambda i: (0, 0)),
      ],
      out_specs=[
          pl.BlockSpec((ROWS_TC, LATENT), lambda i: (i, 0)),
          pl.BlockSpec((ROWS_TC, IN_DIM), lambda i: (i, 0)),
      ],
      out_shape=[
          jax.ShapeDtypeStruct((N, LATENT), jnp.float32),
          jax.ShapeDtypeStruct((N, IN_DIM), jnp.float32),
      ],
  )(s2p, deg, bl2, r2, Wd, bd)


def kernel(x, edge_index, Wl1, bl1, Wr1, Wl2, bl2, Wr2, Wd, bd):
  src = edge_index[0].astype(jnp.int32).reshape(NBLKS, BLK)
  dst = edge_index[1].astype(jnp.int32).reshape(NBLKS, BLK)

  H2 = HIDDEN // 2
  L2 = LATENT // 2
  ones16 = jnp.zeros((BLK, 16), jnp.float32).at[:, 0].set(1.0)
  zh = jnp.zeros((N, H2), jnp.float32)
  zl = jnp.zeros((N, L2), jnp.float32)
  z16 = jnp.zeros((N, 16), jnp.float32)

  p1a, p1b, r1 = _tc_pre(x, Wl1[:H2], Wl1[H2:], Wr1)
  s1p, deg = _segsum_deg(p1a, p1b, src, dst, zh, ones16, z16)
  p2a, p2b, r2 = _tc_mid(s1p, deg, bl1.reshape(1, HIDDEN), r1,
                         Wl2[:L2], Wl2[L2:], Wr2)
  (s2p,) = _segsum_l2(p2a, p2b, src, dst, zl)
  z, x_hat = _tc_post(s2p, deg, bl2.reshape(1, LATENT), r2, Wd,
                      bd.reshape(1, IN_DIM))
  return (z, x_hat)


# speedup vs baseline: 13.7045x; 1.0363x over previous
import jax
import jax.numpy as jnp
from jax import lax
from jax.experimental import pallas as pl
from jax.experimental.pallas import tpu as pltpu
from jax.experimental.pallas import tpu_sc as plsc

N = 10000
E = 320000
IN_DIM = 128
HIDDEN = 128
LATENT = 64

NC = 2
NS = 16
BLK = 128
NBLKS = E // BLK
BASE_BLKS = NBLKS // NS
EXTRA = NBLKS - BASE_BLKS * NS
MAXB = BASE_BLKS + 1
ROWS_A = 624
ROWS_LAST = N - ROWS_A * (NS - 1)

_MESH = plsc.VectorSubcoreMesh(
    core_axis_name="c", subcore_axis_name="s", num_cores=NC, num_subcores=NS)


def _make_segsum(DH, with_deg):
  NBUF = 4
  out_type = [jax.ShapeDtypeStruct((NC, N, DH), jnp.float32)]
  if with_deg:
    out_type.append(jax.ShapeDtypeStruct((NC, N, 16), jnp.float32))
  scratch = [
      pltpu.VMEM((MAXB, BLK), jnp.int32),
      pltpu.VMEM((MAXB, BLK), jnp.int32),
  ]
  scratch += [pltpu.VMEM((BLK, DH), jnp.float32) for _ in range(NBUF)]
  scratch += [
      pltpu.VMEM_SHARED((N, DH), jnp.float32),
  ]
  scratch += [pltpu.SemaphoreType.DMA for _ in range(2 * NBUF)]
  if with_deg:
    scratch += [
        pltpu.VMEM((BLK, 16), jnp.float32),
        pltpu.VMEM_SHARED((N, 16), jnp.float32),
    ]

  def body(*refs):
    if with_deg:
      (p_hbm, ei_hbm, z_hbm, ones_hbm, z1_hbm,
       part_hbm, deg_hbm,
       sidx, didx, *rest) = refs
      bufs = rest[:NBUF]
      acc = rest[NBUF]
      gsems = rest[NBUF + 1:2 * NBUF + 1]
      ssems = rest[2 * NBUF + 1:3 * NBUF + 1]
      ones_v, dacc = rest[3 * NBUF + 1:]
    else:
      (p_hbm, ei_hbm, z_hbm,
       part_hbm,
       sidx, didx, *rest) = refs
      bufs = rest[:NBUF]
      acc = rest[NBUF]
      gsems = rest[NBUF + 1:2 * NBUF + 1]
      ssems = rest[2 * NBUF + 1:3 * NBUF + 1]
    c = lax.axis_index("c")
    s = lax.axis_index("s")
    r0 = s * ROWS_A

    def tile_slices(fn):
      @pl.when(s < NS - 1)
      def _():
        fn(r0, ROWS_A)

      @pl.when(s == NS - 1)
      def _():
        fn(ROWS_A * (NS - 1), ROWS_LAST)

    tile_slices(lambda o, n: pltpu.sync_copy(z_hbm.at[pl.ds(o, n)],
                                             acc.at[pl.ds(o, n)]))
    if with_deg:
      tile_slices(lambda o, n: pltpu.sync_copy(z1_hbm.at[pl.ds(o, n)],
                                               dacc.at[pl.ds(o, n)]))
      pltpu.sync_copy(ones_hbm, ones_v)
    plsc.subcore_barrier()

    nblk = BASE_BLKS + jnp.where(s < EXTRA, 1, 0)
    blk0 = BASE_BLKS * s + jnp.minimum(s, EXTRA)

    @pl.when(s < EXTRA)
    def _():
      pltpu.sync_copy(ei_hbm.at[0, pl.ds(blk0, MAXB)], sidx)
      pltpu.sync_copy(ei_hbm.at[1, pl.ds(blk0, MAXB)], didx)

    @pl.when(s >= EXTRA)
    def _():
      pltpu.sync_copy(ei_hbm.at[0, pl.ds(blk0, BASE_BLKS)],
                      sidx.at[pl.ds(0, BASE_BLKS)])
      pltpu.sync_copy(ei_hbm.at[1, pl.ds(blk0, BASE_BLKS)],
                      didx.at[pl.ds(0, BASE_BLKS)])

    def gather_start(g, b):
      for cc in range(NC):
        @pl.when(c == cc)
        def _(cc=cc):
          pltpu.async_copy(p_hbm.at[cc].at[sidx.at[g]], bufs[b], gsems[b])

    def gather_wait(g, b):
      for cc in range(NC):
        @pl.when(c == cc)
        def _(cc=cc):
          pltpu.make_async_copy(p_hbm.at[cc].at[sidx.at[g]], bufs[b],
                                gsems[b]).wait()

    def scatter_start(g, b):
      pltpu.async_copy(bufs[b], acc.at[didx.at[g]], ssems[b], add=True)

    def scatter_wait(g, b):
      pltpu.make_async_copy(bufs[b], acc.at[didx.at[g]], ssems[b]).wait()

    def deg_issue(g):
      pltpu.sync_copy(ones_v, dacc.at[didx.at[g]], add=True)

    P = NBUF // 2
    NT = BASE_BLKS // NBUF
    for k in range(P):
      gather_start(k, k)

    def round_(j, carry):
      for u in range(NBUF):
        g = j * NBUF + u
        bpre = (u + P) % NBUF
        @pl.when((g + P >= NBUF) & (g + P < nblk))
        def _(g=g, bpre=bpre):
          scatter_wait(g + P - NBUF, bpre)

        @pl.when(g + P < nblk)
        def _(g=g, bpre=bpre):
          gather_start(g + P, bpre)

        gather_wait(g, u)
        scatter_start(g, u)
        if with_deg and (u % 2 == 0):
          @pl.when(c == 0)
          def _(g=g):
            deg_issue(g)

          @pl.when(c == 1)
          def _(g=g):
            deg_issue(g + 1)

      return carry

    lax.fori_loop(0, NT, round_, 0)

    gt = BASE_BLKS
    bt = BASE_BLKS % NBUF

    @pl.when(s < EXTRA)
    def _():
      scatter_wait(gt - NBUF + P, (gt + P) % NBUF)
      gather_wait(gt, bt)
      scatter_start(gt, bt)
      if with_deg:
        @pl.when(c == 0)
        def _():
          deg_issue(gt)

    bw = (gt + P) % NBUF
    for k in range(NBUF):
      g_std = BASE_BLKS - NBUF + k
      if k == bw:
        @pl.when(s >= EXTRA)
        def _(g_std=g_std, k=k):
          scatter_wait(g_std, k)
      elif k == bt:
        @pl.when(s >= EXTRA)
        def _(g_std=g_std, k=k):
          scatter_wait(g_std, k)

        @pl.when(s < EXTRA)
        def _(k=k):
          scatter_wait(gt, k)
      else:
        scatter_wait(g_std, k)

    plsc.subcore_barrier()

    tile_slices(lambda o, n: pltpu.sync_copy(acc.at[pl.ds(o, n)],
                                             part_hbm.at[c, pl.ds(o, n)]))
    if with_deg:
      tile_slices(lambda o, n: pltpu.sync_copy(dacc.at[pl.ds(o, n)],
                                               deg_hbm.at[c, pl.ds(o, n)]))

  params = pltpu.CompilerParams(use_tc_tiling_on_sc=False)
  return pl.kernel(body, out_type=tuple(out_type), mesh=_MESH,
                   scratch_types=scratch, compiler_params=params)


_segsum_deg = _make_segsum(HIDDEN // 2, True)
_segsum_l2 = _make_segsum(LATENT // 2, False)


ROWS_TC = 1000


def _tc_pre(x, Wl1a, Wl1b, Wr1):
  def body(x_ref, wla_ref, wlb_ref, wr_ref, p1_ref, r1_ref):
    xb = x_ref[...]
    dn = (((1,), (1,)), ((), ()))
    p1_ref[0] = lax.dot_general(xb, wla_ref[...], dn,
                                preferred_element_type=jnp.float32)
    p1_ref[1] = lax.dot_general(xb, wlb_ref[...], dn,
                                preferred_element_type=jnp.float32)
    r1_ref[...] = lax.dot_general(xb, wr_ref[...], dn,
                                  preferred_element_type=jnp.float32)
  grid = (N // ROWS_TC,)
  H2 = HIDDEN // 2
  return pl.pallas_call(
      body,
      grid=grid,
      in_specs=[
          pl.BlockSpec((ROWS_TC, IN_DIM), lambda i: (i, 0)),
          pl.BlockSpec((H2, IN_DIM), lambda i: (0, 0)),
          pl.BlockSpec((H2, IN_DIM), lambda i: (0, 0)),
          pl.BlockSpec((HIDDEN, IN_DIM), lambda i: (0, 0)),
      ],
      out_specs=[
          pl.BlockSpec((NC, ROWS_TC, H2), lambda i: (0, i, 0)),
          pl.BlockSpec((ROWS_TC, HIDDEN), lambda i: (i, 0)),
      ],
      out_shape=[
          jax.ShapeDtypeStruct((NC, N, H2), jnp.float32),
          jax.ShapeDtypeStruct((N, HIDDEN), jnp.float32),
      ],
  )(x, Wl1a, Wl1b, Wr1)


def _tc_mid(s1p, deg, bl1, r1, Wl2a, Wl2b, Wr2):
  H2 = HIDDEN // 2
  L2 = LATENT // 2

  def body(s1p_ref, deg_ref, bl1_ref, r1_ref, wla_ref, wlb_ref, wr_ref,
           p2_ref, r2_ref):
    ssum = jnp.concatenate([s1p_ref[0], s1p_ref[1]], axis=1)
    d = jnp.maximum(deg_ref[0, :, 0:1] + deg_ref[1, :, 0:1], 1.0)
    h = jnp.maximum(ssum / d + bl1_ref[...] + r1_ref[...], 0.0)
    dn = (((1,), (1,)), ((), ()))
    p2_ref[0] = lax.dot_general(h, wla_ref[...], dn,
                                preferred_element_type=jnp.float32)
    p2_ref[1] = lax.dot_general(h, wlb_ref[...], dn,
                                preferred_element_type=jnp.float32)
    r2_ref[...] = lax.dot_general(h, wr_ref[...], dn,
                                  preferred_element_type=jnp.float32)
  grid = (N // ROWS_TC,)
  return pl.pallas_call(
      body,
      grid=grid,
      in_specs=[
          pl.BlockSpec((NC, ROWS_TC, H2), lambda i: (0, i, 0)),
          pl.BlockSpec((NC, ROWS_TC, 16), lambda i: (0, i, 0)),
          pl.BlockSpec((1, HIDDEN), lambda i: (0, 0)),
          pl.BlockSpec((ROWS_TC, HIDDEN), lambda i: (i, 0)),
          pl.BlockSpec((L2, HIDDEN), lambda i: (0, 0)),
          pl.BlockSpec((L2, HIDDEN), lambda i: (0, 0)),
          pl.BlockSpec((LATENT, HIDDEN), lambda i: (0, 0)),
      ],
      out_specs=[
          pl.BlockSpec((NC, ROWS_TC, L2), lambda i: (0, i, 0)),
          pl.BlockSpec((ROWS_TC, LATENT), lambda i: (i, 0)),
      ],
      out_shape=[
          jax.ShapeDtypeStruct((NC, N, L2), jnp.float32),
          jax.ShapeDtypeStruct((N, LATENT), jnp.float32),
      ],
  )(s1p, deg, bl1, r1, Wl2a, Wl2b, Wr2)


def _tc_post(s2p, deg, bl2, r2, Wd, bd):
  L2 = LATENT // 2

  def body(s2p_ref, deg_ref, bl2_ref, r2_ref, wd_ref, bd_ref,
           z_ref, xh_ref):
    ssum = jnp.concatenate([s2p_ref[0], s2p_ref[1]], axis=1)
    d = jnp.maximum(deg_ref[0, :, 0:1] + deg_ref[1, :, 0:1], 1.0)
    z = ssum / d + bl2_ref[...] + r2_ref[...]
    z_ref[...] = z
    xh_ref[...] = lax.dot_general(z, wd_ref[...], (((1,), (1,)), ((), ())),
                                  preferred_element_type=jnp.float32) + bd_ref[...]
  grid = (N // ROWS_TC,)
  return pl.pallas_call(
      body,
      grid=grid,
      in_specs=[
          pl.BlockSpec((NC, ROWS_TC, L2), lambda i: (0, i, 0)),
          pl.BlockSpec((NC, ROWS_TC, 16), lambda i: (0, i, 0)),
          pl.BlockSpec((1, LATENT), lambda i: (0, 0)),
          pl.BlockSpec((ROWS_TC, LATENT), lambda i: (i, 0)),
          pl.BlockSpec((IN_DIM, LATENT), lambda i: (0, 0)),
          pl.BlockSpec((1, IN_DIM), lambda i: (0, 0)),
      ],
      out_specs=[
          pl.BlockSpec((ROWS_TC, LATENT), lambda i: (i, 0)),
          pl.BlockSpec((ROWS_TC, IN_DIM), lambda i: (i, 0)),
      ],
      out_shape=[
          jax.ShapeDtypeStruct((N, LATENT), jnp.float32),
          jax.ShapeDtypeStruct((N, IN_DIM), jnp.float32),
      ],
  )(s2p, deg, bl2, r2, Wd, bd)


def kernel(x, edge_index, Wl1, bl1, Wr1, Wl2, bl2, Wr2, Wd, bd):
  ei = edge_index.astype(jnp.int32).reshape(2, NBLKS, BLK)

  H2 = HIDDEN // 2
  L2 = LATENT // 2
  ones16 = jnp.zeros((BLK, 16), jnp.float32).at[:, 0].set(1.0)
  zh = jnp.zeros((N, H2), jnp.float32)
  zl = jnp.zeros((N, L2), jnp.float32)
  z16 = jnp.zeros((N, 16), jnp.float32)

  p1, r1 = _tc_pre(x, Wl1[:H2], Wl1[H2:], Wr1)
  s1p, deg = _segsum_deg(p1, ei, zh, ones16, z16)
  p2, r2 = _tc_mid(s1p, deg, bl1.reshape(1, HIDDEN), r1,
                   Wl2[:L2], Wl2[L2:], Wr2)
  (s2p,) = _segsum_l2(p2, ei, zl)
  z, x_hat = _tc_post(s2p, deg, bl2.reshape(1, LATENT), r2, Wd,
                      bd.reshape(1, IN_DIM))
  return (z, x_hat)
